# baseline (speedup 1.0000x reference)
import jax
import jax.numpy as jnp
from jax.experimental import pallas as pl
from jax.experimental.pallas import tpu as pltpu

N = 4096
D = 128
BLK = 2048
NBLK = N // BLK
BIG = 1000.0


def _normalize(x):
    nrm = jnp.sqrt(jnp.sum(x * x, axis=1, keepdims=True))
    return x / jnp.clip(nrm, 1e-12)


def _fused_body(im_ref, cli_ref, tcol_r_ref, tr_r_ref, tcol_c_ref, tr_c_ref,
                out_ref, imn_ref, clin_ref, racc_ref):
    i = pl.program_id(0)
    big = jnp.bfloat16(BIG)

    @pl.when(i == 0)
    def _init():
        imn_ref[...] = _normalize(im_ref[...]).astype(jnp.bfloat16)
        clin_ref[...] = _normalize(cli_ref[...]).astype(jnp.bfloat16)
        racc_ref[...] = jnp.full((8, N), BIG, dtype=jnp.float32)

    rm0 = jnp.logical_and(tcol_c_ref[...] == 0, tr_c_ref[...] == 1)
    rm1 = tcol_c_ref[...] == 1

    dn = (((1,), (1,)), ((), ()))

    def row_mins(rows_n, cols_n):
        sim = jax.lax.dot_general(rows_n, cols_n, dn,
                                  preferred_element_type=jnp.float32
                                  ).astype(jnp.bfloat16)
        r0 = jnp.min(jnp.where(rm0, sim, big), axis=0, keepdims=True)
        r1 = jnp.min(jnp.where(rm1, sim, big), axis=0, keepdims=True)
        return r0.astype(jnp.float32), r1.astype(jnp.float32)

    imn = imn_ref[...]
    clin = clin_ref[...]
    r0i, r1i = row_mins(imn_ref[pl.ds(i * BLK, BLK), :], imn)
    r0c, r1c = row_mins(clin_ref[pl.ds(i * BLK, BLK), :], clin)
    upd = jnp.concatenate(
        [r0i, r1i, r0c, r1c, jnp.full((4, N), BIG, jnp.float32)], axis=0)
    racc_ref[...] = jnp.minimum(racc_ref[...], upd)

    @pl.when(i == NBLK - 1)
    def _final():
        cm1 = tcol_r_ref[...] == 1
        cm2 = tcol_r_ref[...] == 2
        r0im = racc_ref[0:1, :]
        r1im = racc_ref[1:2, :]
        r0cl = racc_ref[2:3, :]
        r1cl = racc_ref[3:4, :]
        out_ref[0] = jnp.min(jnp.where(cm1, r0im, BIG))
        out_ref[1] = jnp.min(jnp.where(cm2, r0im, BIG))
        out_ref[2] = jnp.min(jnp.where(cm2, r1im, BIG))
        out_ref[3] = jnp.min(jnp.where(cm1, r0cl, BIG))
        out_ref[4] = jnp.min(jnp.where(cm2, r0cl, BIG))
        out_ref[5] = jnp.min(jnp.where(cm2, r1cl, BIG))


def kernel(phi_im, phi_cli, t, traumatic):
    tcol = t[:, 1]
    tcol_row = tcol[None, :].astype(jnp.int32)
    tr_row = traumatic[None, :].astype(jnp.int32)
    tcol_col = tcol[:, None].astype(jnp.int32)
    tr_col = traumatic[:, None].astype(jnp.int32)

    full = lambda shape: pl.BlockSpec(shape, lambda i: (0, 0))
    rowblk = pl.BlockSpec((BLK, 1), lambda i: (i, 0))

    out = pl.pallas_call(
        _fused_body,
        grid=(NBLK,),
        in_specs=[
            full((N, D)),
            full((N, D)),
            full((1, N)),
            full((1, N)),
            rowblk,
            rowblk,
        ],
        out_specs=pl.BlockSpec(memory_space=pltpu.SMEM),
        out_shape=jax.ShapeDtypeStruct((6,), jnp.float32),
        scratch_shapes=[
            pltpu.VMEM((N, D), jnp.bfloat16),
            pltpu.VMEM((N, D), jnp.bfloat16),
            pltpu.VMEM((8, N), jnp.float32),
        ],
        compiler_params=pltpu.CompilerParams(
            dimension_semantics=("arbitrary",),
        ),
    )(phi_im, phi_cli, tcol_row, tr_row, tcol_col, tr_col)
    return out


# speedup vs baseline: 1.0092x; 1.0092x over previous
import jax
import jax.numpy as jnp
from jax.experimental import pallas as pl
from jax.experimental.pallas import tpu as pltpu

N = 4096
D = 128
BLK = 1024
NBLK = N // BLK
BIG = 1000.0


def _normalize(x):
    nrm = jnp.sqrt(jnp.sum(x * x, axis=1, keepdims=True))
    return x / jnp.clip(nrm, 1e-12)


def _fused_body(im_ref, cli_ref, tcol_r_ref, tr_r_ref, tcol_c_ref, tr_c_ref,
                out_ref, imn_ref, clin_ref, racc_ref):
    i = pl.program_id(0)
    big = jnp.bfloat16(BIG)

    @pl.when(i == 0)
    def _init():
        imn_ref[...] = _normalize(im_ref[...]).astype(jnp.bfloat16)
        clin_ref[...] = _normalize(cli_ref[...]).astype(jnp.bfloat16)
        racc_ref[...] = jnp.full((8, N), BIG, dtype=jnp.float32)

    rm0 = jnp.logical_and(tcol_c_ref[...] == 0, tr_c_ref[...] == 1)
    rm1 = tcol_c_ref[...] == 1

    dn = (((1,), (1,)), ((), ()))

    def row_mins(rows_n, cols_n):
        sim = jax.lax.dot_general(rows_n, cols_n, dn,
                                  preferred_element_type=jnp.float32
                                  ).astype(jnp.bfloat16)
        r0 = jnp.min(jnp.where(rm0, sim, big), axis=0, keepdims=True)
        r1 = jnp.min(jnp.where(rm1, sim, big), axis=0, keepdims=True)
        return r0.astype(jnp.float32), r1.astype(jnp.float32)

    imn = imn_ref[...]
    clin = clin_ref[...]
    r0i, r1i = row_mins(imn_ref[pl.ds(i * BLK, BLK), :], imn)
    r0c, r1c = row_mins(clin_ref[pl.ds(i * BLK, BLK), :], clin)
    upd = jnp.concatenate(
        [r0i, r1i, r0c, r1c, jnp.full((4, N), BIG, jnp.float32)], axis=0)
    racc_ref[...] = jnp.minimum(racc_ref[...], upd)

    @pl.when(i == NBLK - 1)
    def _final():
        cm1 = tcol_r_ref[...] == 1
        cm2 = tcol_r_ref[...] == 2
        r0im = racc_ref[0:1, :]
        r1im = racc_ref[1:2, :]
        r0cl = racc_ref[2:3, :]
        r1cl = racc_ref[3:4, :]
        out_ref[0] = jnp.min(jnp.where(cm1, r0im, BIG))
        out_ref[1] = jnp.min(jnp.where(cm2, r0im, BIG))
        out_ref[2] = jnp.min(jnp.where(cm2, r1im, BIG))
        out_ref[3] = jnp.min(jnp.where(cm1, r0cl, BIG))
        out_ref[4] = jnp.min(jnp.where(cm2, r0cl, BIG))
        out_ref[5] = jnp.min(jnp.where(cm2, r1cl, BIG))


def kernel(phi_im, phi_cli, t, traumatic):
    tcol = t[:, 1]
    tcol_row = tcol[None, :].astype(jnp.int32)
    tr_row = traumatic[None, :].astype(jnp.int32)
    tcol_col = tcol[:, None].astype(jnp.int32)
    tr_col = traumatic[:, None].astype(jnp.int32)

    full = lambda shape: pl.BlockSpec(shape, lambda i: (0, 0))
    rowblk = pl.BlockSpec((BLK, 1), lambda i: (i, 0))

    out = pl.pallas_call(
        _fused_body,
        grid=(NBLK,),
        in_specs=[
            full((N, D)),
            full((N, D)),
            full((1, N)),
            full((1, N)),
            rowblk,
            rowblk,
        ],
        out_specs=pl.BlockSpec(memory_space=pltpu.SMEM),
        out_shape=jax.ShapeDtypeStruct((6,), jnp.float32),
        scratch_shapes=[
            pltpu.VMEM((N, D), jnp.bfloat16),
            pltpu.VMEM((N, D), jnp.bfloat16),
            pltpu.VMEM((8, N), jnp.float32),
        ],
        compiler_params=pltpu.CompilerParams(
            dimension_semantics=("arbitrary",),
        ),
    )(phi_im, phi_cli, tcol_row, tr_row, tcol_col, tr_col)
    return out
